# baseline (speedup 1.0000x reference)
import functools

import jax
import jax.numpy as jnp
from jax import lax
from jax.experimental import pallas as pl
from jax.experimental.pallas import tpu as pltpu
from jax.experimental.pallas import tpu_sc as plsc

N = 10000
E = 320000
D = 128
H = 64
GN = 200

NC = 2
NS = 16
NW = NC * NS
EPT = E // NW
W = 80
NWIN = EPT // W
RCHUNK = 200
NRCH = N // RCHUNK

_vmesh = plsc.VectorSubcoreMesh(core_axis_name="c", subcore_axis_name="s")


@functools.partial(
    pl.kernel,
    mesh=_vmesh,
    out_type=jax.ShapeDtypeStruct((E * 80,), jnp.float32),
    scratch_types=[
        pltpu.VMEM((W,), jnp.int32),
        pltpu.VMEM((W,), jnp.int32),
        pltpu.VMEM((W, 128), jnp.float32),
        pltpu.VMEM((W, 128), jnp.float32),
        pltpu.VMEM((W * 64,), jnp.float32),
        pltpu.VMEM((W * 80,), jnp.float32),
    ],
)
def _sc_layer1(xds_hbm, eap_hbm, dst_hbm, src_hbm, out_hbm,
               idxd, idxs, gxd, gxs, gea, hbuf):
    cid = lax.axis_index("c")
    sid = lax.axis_index("s")
    wid = cid * NS + sid
    ones16 = jnp.where(lax.iota(jnp.int32, 16) == 0, 1.0, 0.0)

    @pl.loop(0, NWIN)
    def _(w):
        base = wid * EPT + w * W
        pltpu.sync_copy(dst_hbm.at[pl.ds(base, W)], idxd)
        pltpu.sync_copy(src_hbm.at[pl.ds(base, W)], idxs)
        pltpu.sync_copy(xds_hbm.at[idxd], gxd)
        pltpu.sync_copy(xds_hbm.at[idxs], gxs)
        pltpu.sync_copy(eap_hbm.at[pl.ds(base * 64, W * 64)], gea)

        @pl.loop(0, W)
        def _(r):
            for c in range(4):
                hbuf[pl.ds(r * 80 + c * 16, 16)] = jnp.maximum(
                    gxd[r, pl.ds(c * 16, 16)]
                    + gxs[r, pl.ds(64 + c * 16, 16)]
                    + gea[pl.ds(r * 64 + c * 16, 16)], 0.0)
            hbuf[pl.ds(r * 80 + 64, 16)] = ones16

        pltpu.sync_copy(hbuf, out_hbm.at[pl.ds(base * 80, W * 80)])


@functools.partial(
    pl.kernel,
    mesh=_vmesh,
    out_type=jax.ShapeDtypeStruct((E * 64,), jnp.float32),
    scratch_types=[
        pltpu.VMEM((W,), jnp.int32),
        pltpu.VMEM((W,), jnp.int32),
        pltpu.VMEM((W, 128), jnp.float32),
        pltpu.VMEM((W, 128), jnp.float32),
        pltpu.VMEM((W * 64,), jnp.float32),
    ],
)
def _sc_layer2(pq_hbm, dst_hbm, src_hbm, out_hbm,
               idxd, idxs, gp, gq, hbuf):
    cid = lax.axis_index("c")
    sid = lax.axis_index("s")
    wid = cid * NS + sid

    @pl.loop(0, NWIN)
    def _(w):
        base = wid * EPT + w * W
        pltpu.sync_copy(dst_hbm.at[pl.ds(base, W)], idxd)
        pltpu.sync_copy(src_hbm.at[pl.ds(base, W)], idxs)
        pltpu.sync_copy(pq_hbm.at[idxd], gp)
        pltpu.sync_copy(pq_hbm.at[idxs], gq)

        @pl.loop(0, W)
        def _(r):
            for c in range(4):
                hbuf[pl.ds(r * 64 + c * 16, 16)] = jnp.maximum(
                    gp[r, pl.ds(c * 16, 16)]
                    + gq[r, pl.ds(64 + c * 16, 16)], 0.0)

        pltpu.sync_copy(hbuf, out_hbm.at[pl.ds(base * 64, W * 64)])


EBLK = 8000
NEB = E // EBLK


def _nodeproj_body(x_ref, w_ref, o_ref):
    o_ref[...] = jnp.dot(x_ref[...], w_ref[...],
                         preferred_element_type=jnp.float32)


def _eaproj_body(ea_ref, w_ref, b_ref, o_ref):
    o_ref[...] = jnp.dot(ea_ref[...], w_ref[...],
                         preferred_element_type=jnp.float32) + b_ref[...]


def _segsum_body(h1_ref, dst_ref, o_ref, acc):
    i = pl.program_id(0)

    @pl.when(i == 0)
    def _():
        acc[...] = jnp.zeros((N, 80), jnp.float32)

    def body(e, _):
        idx = dst_ref[0, 0, e]
        acc[pl.ds(idx, 1), :] = (acc[pl.ds(idx, 1), :]
                                 + h1_ref[pl.ds(e, 1), :])
        return 0

    lax.fori_loop(0, EBLK, body, 0)

    @pl.when(i == NEB - 1)
    def _():
        o_ref[...] = acc[...]


def _combine_body(part_ref, xr_ref, w1b_ref, b1b_ref, brt_ref, g1_ref,
                  be1_ref, wp_ref, bpa_ref, wq_ref, p_ref, q_ref):
    s = part_ref[...]
    ssum = s[:, :H]
    deg = s[:, H:H + 1]
    agg = jnp.dot(ssum, w1b_ref[...], preferred_element_type=jnp.float32)
    agg = agg / jnp.maximum(deg, 1.0) + b1b_ref[...] * (deg > 0.0)
    pre = agg + xr_ref[...] + brt_ref[...]
    mu = jnp.mean(pre, axis=0, keepdims=True)
    var = jnp.mean((pre - mu) ** 2, axis=0, keepdims=True)
    x1 = g1_ref[...] * (pre - mu) * lax.rsqrt(var + 1e-5) + be1_ref[...]
    p_ref[...] = jnp.dot(x1, wp_ref[...],
                         preferred_element_type=jnp.float32) + bpa_ref[...]
    q_ref[...] = jnp.dot(x1, wq_ref[...],
                         preferred_element_type=jnp.float32)


def _segmax_body(h2_ref, w2b_ref, b2b_ref, dst_ref, g2_ref, be2_ref,
                 x2_ref, acc, m2):
    i = pl.program_id(0)

    @pl.when(i == 0)
    def _():
        acc[...] = jnp.full((N, H), -jnp.inf, jnp.float32)

    m2[...] = jnp.dot(h2_ref[...], w2b_ref[...],
                      preferred_element_type=jnp.float32) + b2b_ref[...]

    def body(e, _):
        idx = dst_ref[0, 0, e]
        a = acc[pl.ds(idx, 1), :]
        b = m2[pl.ds(e, 1), :]
        acc[pl.ds(idx, 1), :] = jnp.maximum(a, b)
        return 0

    lax.fori_loop(0, EBLK, body, 0)

    @pl.when(i == NEB - 1)
    def _():
        a = acc[...]
        a = jnp.where(a == -jnp.inf, 0.0, a)
        mu = jnp.mean(a, axis=0, keepdims=True)
        var = jnp.mean((a - mu) ** 2, axis=0, keepdims=True)
        x2_ref[...] = (g2_ref[...] * (a - mu) * lax.rsqrt(var + 1e-5)
                       + be2_ref[...])


GBLK = 4
S = N // GN
PROWS = GBLK * S * S
NHB = GN // GBLK


def _pairs_z(x2blk, w3, b3):
    a = x2blk.reshape(GBLK, S, H)
    d = jnp.abs(a[:, :, None, :] - a[:, None, :, :])
    pairs = d.reshape(PROWS, H)
    return jnp.dot(pairs, w3, preferred_element_type=jnp.float32) + b3


def _headstats_body(x2_ref, w3_ref, b3_ref, o_ref, acc):
    i = pl.program_id(0)

    @pl.when(i == 0)
    def _():
        acc[...] = jnp.zeros((8, H), jnp.float32)

    z = _pairs_z(x2_ref[...], w3_ref[...], b3_ref[...])
    acc[0:1, :] += jnp.sum(z, axis=0, keepdims=True)
    acc[1:2, :] += jnp.sum(z * z, axis=0, keepdims=True)

    @pl.when(i == NHB - 1)
    def _():
        o_ref[...] = acc[...]


def _head_body(x2_ref, st_ref, w3_ref, b3_ref, g3_ref, be3_ref, w4_ref,
               b4_ref, o_ref):
    z = _pairs_z(x2_ref[...], w3_ref[...], b3_ref[...])
    total = float(GN * S * S)
    mu = st_ref[0:1, :] / total
    var = st_ref[1:2, :] / total - mu * mu
    h3 = jnp.maximum(g3_ref[...] * (z - mu) * lax.rsqrt(var + 1e-5)
                     + be3_ref[...], 0.0)
    logits = jnp.dot(h3, w4_ref[...],
                     preferred_element_type=jnp.float32) + b4_ref[...]
    m = jnp.max(logits, axis=1, keepdims=True)
    ex = jnp.exp(logits - m)
    lse = m + jnp.log(jnp.sum(ex, axis=1, keepdims=True))
    o_ref[...] = logits - lse


def kernel(x, edge_index, edge_attr, batch, W1a, b1a, W1b, b1b, Wroot, broot,
           g1, be1, W2a, b2a, W2b, b2b, g2, be2, W3, b3, g3, be3, W4, b4):
    f32 = jnp.float32
    dst = edge_index[1]
    src = edge_index[0]

    wnode = jnp.concatenate([W1a[:D], W1a[D:2 * D], Wroot], axis=1)
    nodeproj = pl.pallas_call(
        _nodeproj_body,
        out_shape=jax.ShapeDtypeStruct((N, 3 * H), f32),
    )(x, wnode)
    xds = nodeproj[:, :2 * H]
    xr = nodeproj[:, 2 * H:]

    eap = pl.pallas_call(
        _eaproj_body,
        grid=(16,),
        in_specs=[
            pl.BlockSpec((E // 16, 16), lambda i: (i, 0)),
            pl.BlockSpec((16, H), lambda i: (0, 0)),
            pl.BlockSpec((1, H), lambda i: (0, 0)),
        ],
        out_specs=pl.BlockSpec((E // 16, H), lambda i: (i, 0)),
        out_shape=jax.ShapeDtypeStruct((E, H), f32),
        compiler_params=pltpu.CompilerParams(
            dimension_semantics=("parallel",)),
    )(edge_attr, W1a[2 * D:], b1a.reshape(1, H))

    h1 = _sc_layer1(xds, eap.reshape(E * H), dst, src).reshape(E, 80)

    part = pl.pallas_call(
        _segsum_body,
        grid=(NEB,),
        in_specs=[
            pl.BlockSpec((EBLK, 80), lambda i: (i, 0)),
            pl.BlockSpec((1, 1, EBLK), lambda i: (i, 0, 0),
                         memory_space=pltpu.SMEM),
        ],
        out_specs=pl.BlockSpec((N, 80), lambda i: (0, 0)),
        out_shape=jax.ShapeDtypeStruct((N, 80), f32),
        scratch_shapes=[pltpu.VMEM((N, 80), f32)],
        compiler_params=pltpu.CompilerParams(
            dimension_semantics=("arbitrary",)),
    )(h1, dst.reshape(NEB, 1, EBLK))

    wp = W2a[:H] - W2a[H:]
    wq = W2a[H:]
    p, q = pl.pallas_call(
        _combine_body,
        out_shape=[jax.ShapeDtypeStruct((N, H), f32),
                   jax.ShapeDtypeStruct((N, H), f32)],
    )(part, xr, W1b, b1b.reshape(1, H), broot.reshape(1, H),
      g1.reshape(1, H), be1.reshape(1, H), wp, b2a.reshape(1, H), wq)

    pq = jnp.concatenate([p, q], axis=1)
    h2 = _sc_layer2(pq, dst, src).reshape(E, H)

    x2 = pl.pallas_call(
        _segmax_body,
        grid=(NEB,),
        in_specs=[
            pl.BlockSpec((EBLK, H), lambda i: (i, 0)),
            pl.BlockSpec((H, H), lambda i: (0, 0)),
            pl.BlockSpec((1, H), lambda i: (0, 0)),
            pl.BlockSpec((1, 1, EBLK), lambda i: (i, 0, 0),
                         memory_space=pltpu.SMEM),
            pl.BlockSpec((1, H), lambda i: (0, 0)),
            pl.BlockSpec((1, H), lambda i: (0, 0)),
        ],
        out_specs=pl.BlockSpec((N, H), lambda i: (0, 0)),
        out_shape=jax.ShapeDtypeStruct((N, H), f32),
        scratch_shapes=[pltpu.VMEM((N, H), f32), pltpu.VMEM((EBLK, H), f32)],
        compiler_params=pltpu.CompilerParams(
            dimension_semantics=("arbitrary",)),
    )(h2, W2b, b2b.reshape(1, H), dst.reshape(NEB, 1, EBLK),
      g2.reshape(1, H), be2.reshape(1, H))

    stats = pl.pallas_call(
        _headstats_body,
        grid=(NHB,),
        in_specs=[
            pl.BlockSpec((GBLK * S, H), lambda i: (i, 0)),
            pl.BlockSpec((H, H), lambda i: (0, 0)),
            pl.BlockSpec((1, H), lambda i: (0, 0)),
        ],
        out_specs=pl.BlockSpec((8, H), lambda i: (0, 0)),
        out_shape=jax.ShapeDtypeStruct((8, H), f32),
        scratch_shapes=[pltpu.VMEM((8, H), f32)],
        compiler_params=pltpu.CompilerParams(
            dimension_semantics=("arbitrary",)),
    )(x2, W3, b3.reshape(1, H))

    out = pl.pallas_call(
        _head_body,
        grid=(NHB,),
        in_specs=[
            pl.BlockSpec((GBLK * S, H), lambda i: (i, 0)),
            pl.BlockSpec((8, H), lambda i: (0, 0)),
            pl.BlockSpec((H, H), lambda i: (0, 0)),
            pl.BlockSpec((1, H), lambda i: (0, 0)),
            pl.BlockSpec((1, H), lambda i: (0, 0)),
            pl.BlockSpec((1, H), lambda i: (0, 0)),
            pl.BlockSpec((H, 2), lambda i: (0, 0)),
            pl.BlockSpec((1, 2), lambda i: (0, 0)),
        ],
        out_specs=pl.BlockSpec((PROWS, 2), lambda i: (i, 0)),
        out_shape=jax.ShapeDtypeStruct((GN * S * S, 2), f32),
        compiler_params=pltpu.CompilerParams(
            dimension_semantics=("arbitrary",)),
    )(x2, stats, W3, b3.reshape(1, H), g3.reshape(1, H), be3.reshape(1, H),
      W4, b4.reshape(1, 2))

    return out
